# baseline (speedup 1.0000x reference)
import functools

import jax
import jax.numpy as jnp
from jax import lax
from jax.experimental import pallas as pl
from jax.experimental.pallas import tpu as pltpu
from jax.experimental.pallas import tpu_sc as plsc

N = 10000
E = 320000
D = 128
H = 64
L = 3
G = 128

NQ = N // 4
NC = 2
NS = 16
NW = NC * NS
C = 128
CH = 80
KF = 5
NGRP = CH // KF
E_PAD = NW * CH * C
N_PAD = 10240
ROWS_PER_TILE = N_PAD // NS


def _make_sc_segsum():
    mesh = plsc.VectorSubcoreMesh(core_axis_name="c", subcore_axis_name="s")

    @functools.partial(
        pl.kernel,
        out_type=jax.ShapeDtypeStruct((NC, N_PAD, H), jnp.bfloat16),
        mesh=mesh,
        scratch_types=[
            pltpu.VMEM((CH, C), jnp.int32),
            pltpu.VMEM((CH, C), jnp.int32),
            pltpu.VMEM((KF, C, H), jnp.bfloat16),
            pltpu.VMEM_SHARED((N_PAD, H), jnp.bfloat16),
            pltpu.SemaphoreType.DMA((KF,)),
        ],
        compiler_params=pltpu.CompilerParams(use_tc_tiling_on_sc=False),
    )
    def sc_segsum(y_hbm, ei_hbm, zero_hbm, out_hbm,
                  src_v, dst_v, rows_v, agg_sh, sem):
        c = lax.axis_index("c")
        s = lax.axis_index("s")
        wid = s * NC + c
        row0 = pl.multiple_of(s * ROWS_PER_TILE, 8)

        pltpu.sync_copy(ei_hbm.at[0, pl.ds(wid * CH, CH)], src_v)
        pltpu.sync_copy(ei_hbm.at[1, pl.ds(wid * CH, CH)], dst_v)

        pltpu.sync_copy(zero_hbm.at[pl.ds(row0, ROWS_PER_TILE)],
                        agg_sh.at[pl.ds(row0, ROWS_PER_TILE)])
        plsc.subcore_barrier()

        def fire(g, b):
            pltpu.async_copy(y_hbm.at[src_v.at[g * KF + b]], rows_v.at[b],
                             sem.at[b])

        def drain(g, b):
            pltpu.make_async_copy(y_hbm.at[src_v.at[g * KF + b]],
                                  rows_v.at[b], sem.at[b]).wait()
            pltpu.sync_copy(rows_v.at[b], agg_sh.at[dst_v.at[g * KF + b]],
                            add=True)

        for b in range(KF):
            fire(0, b)

        def group(g, _):
            for b in range(KF):
                drain(g - 1, b)
                fire(g, b)
            return 0

        lax.fori_loop(1, NGRP, group, 0)
        for b in range(KF):
            drain(NGRP - 1, b)
        plsc.subcore_barrier()

        pltpu.sync_copy(agg_sh.at[pl.ds(row0, ROWS_PER_TILE)],
                        out_hbm.at[c, pl.ds(row0, ROWS_PER_TILE)])

    return sc_segsum


_sc_segsum = _make_sc_segsum()


def _pack_table(y4):
    lo = jnp.concatenate([y4[:, 64 * t: 64 * t + 32] for t in range(4)],
                         axis=1)
    hi = jnp.concatenate([y4[:, 64 * t + 32: 64 * t + 64] for t in range(4)],
                         axis=1)
    lo16 = lax.bitcast_convert_type(
        lo.astype(jnp.bfloat16), jnp.uint16).astype(jnp.uint32)
    hi16 = lax.bitcast_convert_type(
        hi.astype(jnp.bfloat16), jnp.uint16).astype(jnp.uint32)
    return lax.bitcast_convert_type(lo16 | (hi16 << 16), jnp.float32)


def _unpack_agg(agg_ref):
    def halves(p):
        u = lax.bitcast_convert_type(p, jnp.uint32)
        lo = lax.bitcast_convert_type(
            u.astype(jnp.uint16), jnp.bfloat16).astype(jnp.float32)
        hi = lax.bitcast_convert_type(
            (u >> 16).astype(jnp.uint16), jnp.bfloat16).astype(jnp.float32)
        return lo, hi

    lo0, hi0 = halves(agg_ref[0, :NQ])
    lo1, hi1 = halves(agg_ref[1, :NQ])
    lo = lo0 + lo1
    hi = hi0 + hi1
    return jnp.concatenate(
        [jnp.concatenate([lo[:, 32 * t: 32 * t + 32],
                          hi[:, 32 * t: 32 * t + 32]], axis=1)
         for t in range(4)], axis=1)


def _mm0_body(x4_ref, w_ref, o_ref, ypk_ref):
    y4 = jnp.dot(x4_ref[...], w_ref[...], preferred_element_type=jnp.float32)
    o_ref[...] = y4
    ypk_ref[...] = _pack_table(y4)


def _pool(h4, bt_refs):
    iota = lax.broadcasted_iota(jnp.int32, (G, NQ), 0)
    acc = None
    for t in range(4):
        oh = (iota == bt_refs[t][...]).astype(jnp.float32)
        m = jnp.dot(oh, h4, preferred_element_type=jnp.float32)
        part = m[:, 64 * t: 64 * t + 64]
        acc = part if acc is None else acc + part
    return acc


def _layer_body(y_ref, agg_ref, scale_ref, b1_ref, w2_ref, b2_ref,
                wn_ref, b0_ref, bt1_ref, bt2_ref, bt3_ref,
                ynext_ref, ypk_ref, pooled_ref):
    t = scale_ref[...] * y_ref[...] + _unpack_agg(agg_ref) + b1_ref[...]
    u = jnp.maximum(t, 0.0)
    h4 = jnp.maximum(
        jnp.dot(u, w2_ref[...], preferred_element_type=jnp.float32)
        + b2_ref[...], 0.0)
    yn = jnp.dot(h4, wn_ref[...], preferred_element_type=jnp.float32)
    ynext_ref[...] = yn
    ypk_ref[...] = _pack_table(yn)
    pooled_ref[...] = _pool(h4, (b0_ref, bt1_ref, bt2_ref, bt3_ref))


def _layer_last_body(y_ref, agg_ref, scale_ref, b1_ref, w2_ref, b2_ref,
                     b0_ref, bt1_ref, bt2_ref, bt3_ref, pooled_ref):
    t = scale_ref[...] * y_ref[...] + _unpack_agg(agg_ref) + b1_ref[...]
    u = jnp.maximum(t, 0.0)
    h4 = jnp.maximum(
        jnp.dot(u, w2_ref[...], preferred_element_type=jnp.float32)
        + b2_ref[...], 0.0)
    pooled_ref[...] = _pool(h4, (b0_ref, bt1_ref, bt2_ref, bt3_ref))


def _blockdiag4(w):
    din, dout = w.shape
    z = jnp.zeros((din, dout), jnp.float32)
    rows = []
    for t in range(4):
        rows.append(jnp.concatenate(
            [w if i == t else z for i in range(4)], axis=1))
    return jnp.concatenate(rows, axis=0)


def _dup4(b):
    return jnp.concatenate([b, b, b, b]).reshape(1, 4 * H)


def kernel(x, edge_index, batch, params, eps, center):
    npad = E_PAD - E
    k = jnp.arange(npad, dtype=jnp.int32)
    pads = jnp.stack([k % N, N + (k % (N_PAD - N))])
    ei = jnp.concatenate([edge_index, pads], axis=1).reshape(2, NW * CH, C)

    zeros = jnp.zeros((N_PAD, H), jnp.bfloat16)
    x4 = x.reshape(NQ, 4 * D)
    bts = [batch[t::4].reshape(1, NQ) for t in range(4)]

    def _as_bf16_table(ypk):
        return lax.bitcast_convert_type(
            ypk.reshape(N * H // 2), jnp.bfloat16).reshape(N, H)

    def _as_f32_packed(agg):
        return lax.bitcast_convert_type(
            agg.reshape(NC, N_PAD * H // 2, 2),
            jnp.float32).reshape(NC, N_PAD // 4, 2 * H)

    y4, ypk = pl.pallas_call(
        _mm0_body,
        out_shape=(jax.ShapeDtypeStruct((NQ, 4 * H), jnp.float32),
                   jax.ShapeDtypeStruct((NQ, 2 * H), jnp.float32)),
    )(x4, _blockdiag4(params[0][0]))

    pooled = []
    for l in range(L):
        W1, b1, W2, b2 = params[l]
        agg = _as_f32_packed(_sc_segsum(_as_bf16_table(ypk), ei, zeros))
        scale = (1.0 + eps[l]).reshape(1, 1)
        if l + 1 < L:
            y4, ypk, p = pl.pallas_call(
                _layer_body,
                out_shape=(jax.ShapeDtypeStruct((NQ, 4 * H), jnp.float32),
                           jax.ShapeDtypeStruct((NQ, 2 * H), jnp.float32),
                           jax.ShapeDtypeStruct((G, H), jnp.float32)),
            )(y4, agg, scale, _dup4(b1), _blockdiag4(W2), _dup4(b2),
              _blockdiag4(params[l + 1][0]), *bts)
        else:
            p = pl.pallas_call(
                _layer_last_body,
                out_shape=jax.ShapeDtypeStruct((G, H), jnp.float32),
            )(y4, agg, scale, _dup4(b1), _blockdiag4(W2), _dup4(b2), *bts)
        pooled.append(p)

    z = jnp.concatenate(pooled, axis=-1)
    return (z, center)
